# baseline (speedup 1.0000x reference)
import jax
import jax.numpy as jnp
from jax.experimental import pallas as pl
from jax.experimental.pallas import tpu as pltpu

_VOCAB = 1000
_EMB = 64
_RBLK = 512


def _tc_body(x_ref, w_ref, o_ref):
    xb = x_ref[...]
    mx = jnp.max(xb, axis=1, keepdims=True)
    iota = jax.lax.broadcasted_iota(jnp.int32, xb.shape, 1)
    cand = jnp.where(xb == mx, iota, jnp.int32(_VOCAB))
    tok = jnp.min(cand, axis=1, keepdims=True)
    onehot = (iota == tok).astype(jnp.float32)
    o_ref[...] = jax.lax.dot_general(
        onehot, w_ref[...], (((1,), (0,)), ((), ())),
        preferred_element_type=jnp.float32)


def kernel(x, W):
    B, N, V = x.shape
    R = B * N
    x2 = x.reshape(R, V)
    grid = (R // _RBLK,)
    out = pl.pallas_call(
        _tc_body,
        grid=grid,
        in_specs=[
            pl.BlockSpec((_RBLK, V), lambda i: (i, 0)),
            pl.BlockSpec((V, _EMB), lambda i: (0, 0)),
        ],
        out_specs=pl.BlockSpec((_RBLK, _EMB), lambda i: (i, 0)),
        out_shape=jax.ShapeDtypeStruct((R, _EMB), jnp.float32),
        compiler_params=pltpu.CompilerParams(
            dimension_semantics=("arbitrary",),
        ),
    )(x2, W)
    return out.reshape(B, N, _EMB)
